# baseline (speedup 1.0000x reference)
import functools

import jax
import jax.numpy as jnp
from jax import lax
from jax.experimental import pallas as pl
from jax.experimental.pallas import tpu as pltpu
from jax.experimental.pallas import tpu_sc as plsc

_N_ELEMENTS = 1000000
_D = 16
_B = 16384

_NC = 2
_NS = 16
_NW = _NC * _NS
_BPW = _B // _NW


@functools.partial(
    pl.kernel,
    mesh=plsc.VectorSubcoreMesh(core_axis_name="c", subcore_axis_name="s"),
    compiler_params=pltpu.CompilerParams(use_tc_tiling_on_sc=False),
    out_type=[
        jax.ShapeDtypeStruct((_B, _D), jnp.float32),
        jax.ShapeDtypeStruct((_B, _D), jnp.float32),
    ],
    scratch_types=[
        pltpu.VMEM((_BPW,), jnp.int32),
        pltpu.VMEM((_BPW, _D), jnp.float32),
        pltpu.VMEM((_BPW, _D), jnp.float32),
        pltpu.SemaphoreType.DMA,
        pltpu.SemaphoreType.DMA,
    ],
)
def _sc_gather(idx_hbm, mean_hbm, rawvar_hbm, out_m, out_v,
               idx_v, rows_m, rows_v, sem_m, sem_v):
    wid = lax.axis_index("s") * _NC + lax.axis_index("c")
    base = wid * _BPW
    pltpu.sync_copy(idx_hbm.at[pl.ds(base, _BPW)], idx_v)
    cp_m = pltpu.async_copy(mean_hbm.at[idx_v], rows_m, sem_m)
    cp_v = pltpu.async_copy(rawvar_hbm.at[idx_v], rows_v, sem_v)
    cp_m.wait()
    pltpu.sync_copy(rows_m, out_m.at[pl.ds(base, _BPW)])
    cp_v.wait()
    pltpu.sync_copy(rows_v, out_v.at[pl.ds(base, _BPW)])


def _softplus_stack_body(m_ref, v_ref, o_ref):
    o_ref[0] = m_ref[:]
    x = v_ref[:]
    o_ref[1] = jnp.maximum(x, 0.0) + jnp.log1p(jnp.exp(-jnp.abs(x)))


_ROWS128 = _B * _D // 128


def _softplus_stack(m2, v2):
    return pl.pallas_call(
        _softplus_stack_body,
        out_shape=jax.ShapeDtypeStruct((2, _ROWS128, 128), jnp.float32),
    )(m2, v2)


def kernel(indices, variational_mean, raw_variational_variance):
    idx = indices.astype(jnp.int32)
    ms, vs_raw = _sc_gather(idx, variational_mean, raw_variational_variance)
    out = _softplus_stack(ms.reshape(_ROWS128, 128), vs_raw.reshape(_ROWS128, 128))
    return out.reshape(2, _B, _D)
